# baseline (speedup 1.0000x reference)
import jax
import jax.numpy as jnp
from jax.experimental import pallas as pl
from jax.experimental.pallas import tpu as pltpu

EMB = 128
NP = 100
NC = 500
N = 50000
TILE = 2048
GRID = -(-N // TILE)
F32 = jnp.float32
BF16 = jnp.bfloat16


def _dot(a, b):
    return jax.lax.dot_general(a.astype(BF16), b.astype(BF16),
                               (((1,), (0,)), ((), ())),
                               preferred_element_type=F32)


def _rowsum_lanes(x):
    ones = jnp.ones((1, EMB), BF16)
    return jax.lax.dot_general(ones, x.astype(BF16), (((1,), (1,)), ((), ())),
                               preferred_element_type=F32)


def _intra_small(adj, m, emb2):
    s = _rowsum_lanes(emb2)
    mn = jnp.min(s, axis=(0, 1), keepdims=True)
    mx = jnp.max(s, axis=(0, 1), keepdims=True)
    shift = jnp.where(m >= 0.0, m * mx, m * mn)
    p = jnp.exp(m * s - shift)
    w = adj * p
    num = _dot(w, emb2)
    den = jnp.sum(w, axis=1, keepdims=True)
    z = jnp.sum(p, axis=1, keepdims=True)
    return num / (den + 1e-8 * z)


def _inter_small(e1, e2, e3, a1, a2, a3, w1, w2, b):
    gate = jax.nn.sigmoid(_dot(e1, a1) + _dot(e2, a2) + _dot(e3, a3)
                          + _dot(e2, w1) + _dot(e3, w2) + b)
    return e1 + gate * e2 + (1.0 - gate) * e3


def _mega_body(e_ref, advp_ref, advc_ref, adpv_ref, adcv_ref,
               pe_ref, ce_ref, mpv_ref, mcv_ref,
               ga1_ref, ga2_ref, ga3_ref, g1_ref, g2_ref, gb_ref,
               adpc_ref, adcp_ref, mpc_ref, mcp_ref,
               pa1_ref, pa2_ref, pa3_ref, p1_ref, p2_ref, pb_ref,
               ca1_ref, ca2_ref, ca3_ref, c1_ref, c2_ref, cb_ref,
               item_ref, numpv_ref, denpv_ref, zpv_ref,
               numcv_ref, dencv_ref, zcv_ref, price1_ref, cate1_ref,
               item_sc, pe_sc, ce_sc, shpv_sc, shcv_sc):
    l = pl.program_id(0)
    i = pl.program_id(1)

    @pl.when((l == 0) & (i == 0))
    def _():
        pe_sc[...] = pe_ref[...]
        ce_sc[...] = ce_ref[...]

    @pl.when((l == 1) & (i == 0))
    def _():
        pe0 = pe_sc[...]
        ce0 = ce_sc[...]
        e_pv = numpv_ref[...] / (denpv_ref[...] + 1e-8 * zpv_ref[...])
        e_cv = numcv_ref[...] / (dencv_ref[...] + 1e-8 * zcv_ref[...])
        e_pc = _intra_small(adpc_ref[...], mpc_ref[...], ce0)
        e_cp = _intra_small(adcp_ref[...], mcp_ref[...], pe0)
        price = _inter_small(pe0, e_pv, e_pc, pa1_ref[...], pa2_ref[...],
                             pa3_ref[...], p1_ref[...], p2_ref[...],
                             pb_ref[...])
        cate = _inter_small(ce0, e_cp, e_cv, ca1_ref[...], ca2_ref[...],
                            ca3_ref[...], c1_ref[...], c2_ref[...],
                            cb_ref[...])
        pe_sc[...] = price
        ce_sc[...] = cate
        price1_ref[...] = price
        cate1_ref[...] = cate

    @pl.when(i == 0)
    def _():
        for ref in (numpv_ref, denpv_ref, zpv_ref, numcv_ref, dencv_ref,
                    zcv_ref):
            ref[...] = jnp.zeros(ref.shape, ref.dtype)
        shpv_sc[...] = jnp.full(shpv_sc.shape, -jnp.inf, F32)
        shcv_sc[...] = jnp.full(shcv_sc.shape, -jnp.inf, F32)

    def body(masked):
        e_l0 = e_ref[...]
        e_l1 = item_sc[pl.ds(i * TILE, TILE), :].astype(F32)
        e = jnp.where(l == 0, e_l0, e_l1)
        if masked:
            rows = jax.lax.broadcasted_iota(jnp.int32, (TILE, 1), 0) + i * TILE
            lanes = jax.lax.broadcasted_iota(jnp.int32, (1, TILE), 1) + i * TILE
            rmask, lmask = rows < N, lanes < N
            e = jnp.where(rmask, e, 0.0)
        pe = pe_sc[...]
        ce = ce_sc[...]
        evp = _dot(advp_ref[...], pe)
        evc = _dot(advc_ref[...], ce)
        garg = (_dot(e, ga1_ref[...]) + _dot(evp, ga2_ref[...])
                + _dot(evc, ga3_ref[...]) + _dot(evp, g1_ref[...])
                + _dot(evc, g2_ref[...]) + gb_ref[...])
        g = jax.nn.sigmoid(garg)
        item = e + g * evp + (1.0 - g) * evc

        @pl.when(l == 0)
        def _():
            item_sc[pl.ds(i * TILE, TILE), :] = item.astype(BF16)

        @pl.when(l == 1)
        def _():
            item_ref[...] = item

        s = _rowsum_lanes(e)
        if masked:
            smx = jnp.max(jnp.where(lmask, s, -jnp.inf), axis=(0, 1),
                          keepdims=True)
            smn = jnp.min(jnp.where(lmask, s, jnp.inf), axis=(0, 1),
                          keepdims=True)
        else:
            smx = jnp.max(s, axis=(0, 1), keepdims=True)
            smn = jnp.min(s, axis=(0, 1), keepdims=True)

        def intra_part(m, adj, num_ref, den_ref, z_ref, sh_ref):
            sh_t = jnp.where(m >= 0.0, m * smx, m * smn)
            p = jnp.exp(m * s - sh_t)
            if masked:
                p = jnp.where(lmask, p, 0.0)
                w = jnp.where(lmask, adj * p, 0.0)
            else:
                w = adj * p
            sh_old = sh_ref[...]
            sh_new = jnp.maximum(sh_old, sh_t)
            alpha = jnp.exp(sh_old - sh_new)
            beta = jnp.exp(sh_t - sh_new)
            num_ref[...] = num_ref[...] * alpha + _dot(w, e) * beta
            den_ref[...] = (den_ref[...] * alpha
                            + jnp.sum(w, axis=1, keepdims=True) * beta)
            z_ref[...] = (z_ref[...] * alpha
                          + jnp.sum(p, axis=1, keepdims=True) * beta)
            sh_ref[...] = sh_new

        intra_part(mpv_ref[...], adpv_ref[...],
                   numpv_ref, denpv_ref, zpv_ref, shpv_sc)
        intra_part(mcv_ref[...], adcv_ref[...],
                   numcv_ref, dencv_ref, zcv_ref, shcv_sc)

    @pl.when(i < GRID - 1)
    def _():
        body(masked=False)

    @pl.when(i == GRID - 1)
    def _():
        body(masked=True)


def _mega(e, advp, advc, adpv, adcv, pe, ce, mpv, mcv,
          ga1, ga2, ga3, g1, g2, gb, adpc, adcp, mpc, mcp, pw, cw):
    c = lambda *shape: pl.BlockSpec(shape, lambda l, i: (0,) * len(shape))
    return pl.pallas_call(
        _mega_body,
        grid=(2, GRID),
        in_specs=[
            pl.BlockSpec((TILE, EMB), lambda l, i: (i * (1 - l), 0)),
            pl.BlockSpec((TILE, NP), lambda l, i: (i, 0)),
            pl.BlockSpec((TILE, NC), lambda l, i: (i, 0)),
            pl.BlockSpec((NP, TILE), lambda l, i: (0, i)),
            pl.BlockSpec((NC, TILE), lambda l, i: (0, i)),
            c(NP, EMB), c(NC, EMB), c(NP, 1), c(NC, 1),
            c(EMB, EMB), c(EMB, EMB), c(EMB, EMB), c(EMB, EMB), c(EMB, EMB),
            c(1, EMB),
            c(NP, NC), c(NC, NP), c(NP, 1), c(NC, 1),
            c(EMB, EMB), c(EMB, EMB), c(EMB, EMB), c(EMB, EMB), c(EMB, EMB),
            c(1, EMB),
            c(EMB, EMB), c(EMB, EMB), c(EMB, EMB), c(EMB, EMB), c(EMB, EMB),
            c(1, EMB),
        ],
        out_specs=[
            pl.BlockSpec((TILE, EMB), lambda l, i: (i * l, 0)),
            c(NP, EMB), c(NP, 1), c(NP, 1),
            c(NC, EMB), c(NC, 1), c(NC, 1),
            c(NP, EMB), c(NC, EMB),
        ],
        out_shape=[
            jax.ShapeDtypeStruct((N, EMB), F32),
            jax.ShapeDtypeStruct((NP, EMB), F32),
            jax.ShapeDtypeStruct((NP, 1), F32),
            jax.ShapeDtypeStruct((NP, 1), F32),
            jax.ShapeDtypeStruct((NC, EMB), F32),
            jax.ShapeDtypeStruct((NC, 1), F32),
            jax.ShapeDtypeStruct((NC, 1), F32),
            jax.ShapeDtypeStruct((NP, EMB), F32),
            jax.ShapeDtypeStruct((NC, EMB), F32),
        ],
        scratch_shapes=[
            pltpu.VMEM((GRID * TILE, EMB), BF16),
            pltpu.VMEM((NP, EMB), F32),
            pltpu.VMEM((NC, EMB), F32),
            pltpu.VMEM((NP, 1), F32),
            pltpu.VMEM((NC, 1), F32),
        ],
    )(e, advp, advc, adpv, adcv, pe, ce, mpv, mcv,
      ga1, ga2, ga3, g1, g2, gb, adpc, adcp, mpc, mcp, *pw, *cw)


def _finalize_body(pe_ref, ce_ref, numpv_ref, denpv_ref, zpv_ref,
                   numcv_ref, dencv_ref, zcv_ref, adpc_ref, adcp_ref,
                   mpc_ref, mcp_ref,
                   pa1_ref, pa2_ref, pa3_ref, p1_ref, p2_ref, pb_ref,
                   ca1_ref, ca2_ref, ca3_ref, c1_ref, c2_ref, cb_ref,
                   price_ref, cate_ref):
    pe = pe_ref[...]
    ce = ce_ref[...]
    e_pv = numpv_ref[...] / (denpv_ref[...] + 1e-8 * zpv_ref[...])
    e_cv = numcv_ref[...] / (dencv_ref[...] + 1e-8 * zcv_ref[...])
    e_pc = _intra_small(adpc_ref[...], mpc_ref[...], ce)
    e_cp = _intra_small(adcp_ref[...], mcp_ref[...], pe)
    price_ref[...] = _inter_small(pe, e_pv, e_pc, pa1_ref[...], pa2_ref[...],
                                  pa3_ref[...], p1_ref[...], p2_ref[...],
                                  pb_ref[...])
    cate_ref[...] = _inter_small(ce, e_cp, e_cv, ca1_ref[...], ca2_ref[...],
                                 ca3_ref[...], c1_ref[...], c2_ref[...],
                                 cb_ref[...])


def _finalize(pe, ce, numpv, denpv, zpv, numcv, dencv, zcv, adpc, adcp,
              mpc, mcp, pw, cw):
    return pl.pallas_call(
        _finalize_body,
        out_shape=[jax.ShapeDtypeStruct((NP, EMB), F32),
                   jax.ShapeDtypeStruct((NC, EMB), F32)],
    )(pe, ce, numpv, denpv, zpv, numcv, dencv, zcv, adpc, adcp,
      mpc, mcp, *pw, *cw)


def kernel(adjacency, adjacency_pv, adjacency_vp, adjacency_pc, adjacency_cp,
           adjacency_cv, adjacency_vc, embedding, pri_emb, cate_emb,
           single_basket, session_basket, mat_pv, mat_pc, mat_cp, mat_cv,
           W_aogi, b_aogi, W_bgi1, b_bgi1, W_bgi2, b_bgi2,
           W_aogp, b_aogp, W_bgp1, b_bgp1, W_bgp2, b_bgp2,
           W_aogc, b_aogc, W_bgc1, b_bgc1, W_bgc2, b_bgc2):
    def split(Wa, ba, W1, b1, W2, b2):
        return (Wa[:EMB], Wa[EMB:2 * EMB], Wa[2 * EMB:], W1, W2,
                (ba + b1 + b2)[None, :])

    gi = split(W_aogi, b_aogi, W_bgi1, b_bgi1, W_bgi2, b_bgi2)
    gp = split(W_aogp, b_aogp, W_bgp1, b_bgp1, W_bgp2, b_bgp2)
    gc = split(W_aogc, b_aogc, W_bgc1, b_bgc1, W_bgc2, b_bgc2)

    rd = lambda m: m.astype(BF16).astype(F32)
    mpv, mpc, mcp, mcv = rd(mat_pv), rd(mat_pc), rd(mat_cp), rd(mat_cv)

    (item2, numpv, denpv, zpv, numcv, dencv, zcv, price1, cate1) = _mega(
        embedding, adjacency_vp, adjacency_vc, adjacency_pv, adjacency_cv,
        pri_emb, cate_emb, mpv, mcv, *gi,
        adjacency_pc, adjacency_cp, mpc, mcp, gp, gc)
    price2, cate2 = _finalize(price1, cate1, numpv, denpv, zpv,
                              numcv, dencv, zcv,
                              adjacency_pc, adjacency_cp, mpc, mcp, gp, gc)
    return (item2, price2, cate2)


# speedup vs baseline: 1.1273x; 1.1273x over previous
import jax
import jax.numpy as jnp
from jax.experimental import pallas as pl

EMB = 128
NP = 100
NC = 500
N = 50000
TILE = 2048
GRID = -(-N // TILE)
F32 = jnp.float32
BF16 = jnp.bfloat16


def _bdot(a, b):
    return jax.lax.dot_general(a.astype(BF16), b.astype(BF16),
                               (((1,), (0,)), ((), ())),
                               preferred_element_type=F32)


def _rowsum_lanes(x):
    ones = jnp.ones((1, EMB), BF16)
    return jax.lax.dot_general(ones, x.astype(BF16), (((1,), (1,)), ((), ())),
                               preferred_element_type=F32)


def _masks(i):
    rows = jax.lax.broadcasted_iota(jnp.int32, (TILE, 1), 0) + i * TILE
    lanes = jax.lax.broadcasted_iota(jnp.int32, (1, TILE), 1) + i * TILE
    return rows < N, lanes < N


def _prepass_body(e_ref, mn_ref, mx_ref):
    i = pl.program_id(0)
    rmask, lmask = _masks(i)
    s = _rowsum_lanes(jnp.where(rmask, e_ref[...], 0.0))
    mn_t = jnp.min(jnp.where(lmask, s, jnp.inf), axis=(0, 1), keepdims=True)
    mx_t = jnp.max(jnp.where(lmask, s, -jnp.inf), axis=(0, 1), keepdims=True)

    @pl.when(i == 0)
    def _():
        mn_ref[...] = mn_t
        mx_ref[...] = mx_t

    @pl.when(i > 0)
    def _():
        mn_ref[...] = jnp.minimum(mn_ref[...], mn_t)
        mx_ref[...] = jnp.maximum(mx_ref[...], mx_t)


def _prepass(e):
    return pl.pallas_call(
        _prepass_body,
        grid=(GRID,),
        in_specs=[pl.BlockSpec((TILE, EMB), lambda i: (i, 0))],
        out_specs=[pl.BlockSpec((1, 1), lambda i: (0, 0)),
                   pl.BlockSpec((1, 1), lambda i: (0, 0))],
        out_shape=[jax.ShapeDtypeStruct((1, 1), F32),
                   jax.ShapeDtypeStruct((1, 1), F32)],
    )(e)


def _layer_body(e_ref, advp_ref, advc_ref, adpv_ref, adcv_ref,
                pe_ref, ce_ref, mpv_ref, mcv_ref, mn_ref, mx_ref,
                ga1_ref, ga2_ref, ga3_ref, g1_ref, g2_ref, gb_ref,
                item_ref, numpv_ref, denpv_ref, zpv_ref,
                numcv_ref, dencv_ref, zcv_ref, mnn_ref, mxn_ref):
    i = pl.program_id(0)

    def body(masked):
        if masked:
            rmask, lmask = _masks(i)
            e = jnp.where(rmask, e_ref[...], 0.0)
        else:
            e = e_ref[...]
        evp = _bdot(advp_ref[...], pe_ref[...])
        evc = _bdot(advc_ref[...], ce_ref[...])
        garg = (_bdot(e, ga1_ref[...]) + _bdot(evp, ga2_ref[...])
                + _bdot(evc, ga3_ref[...]) + _bdot(evp, g1_ref[...])
                + _bdot(evc, g2_ref[...]) + gb_ref[...])
        g = jax.nn.sigmoid(garg)
        item = e + g * evp + (1.0 - g) * evc
        item_ref[...] = item

        s = _rowsum_lanes(e)
        mn_s = mn_ref[...]
        mx_s = mx_ref[...]

        def intra_part(m, adj):
            shift = jnp.where(m >= 0.0, m * mx_s, m * mn_s)
            p = jnp.exp(m * s - shift)
            if masked:
                p = jnp.where(lmask, p, 0.0)
                w = jnp.where(lmask, adj * p, 0.0)
            else:
                w = adj * p
            num = _bdot(w, e)
            den = jnp.sum(w, axis=1, keepdims=True)
            z = jnp.sum(p, axis=1, keepdims=True)
            return num, den, z

        numpv, denpv, zpv = intra_part(mpv_ref[...], adpv_ref[...])
        numcv, dencv, zcv = intra_part(mcv_ref[...], adcv_ref[...])

        s_next = _rowsum_lanes(item)
        if masked:
            mn_t = jnp.min(jnp.where(lmask, s_next, jnp.inf),
                           axis=(0, 1), keepdims=True)
            mx_t = jnp.max(jnp.where(lmask, s_next, -jnp.inf),
                           axis=(0, 1), keepdims=True)
        else:
            mn_t = jnp.min(s_next, axis=(0, 1), keepdims=True)
            mx_t = jnp.max(s_next, axis=(0, 1), keepdims=True)

        @pl.when(i == 0)
        def _():
            numpv_ref[...] = numpv
            denpv_ref[...] = denpv
            zpv_ref[...] = zpv
            numcv_ref[...] = numcv
            dencv_ref[...] = dencv
            zcv_ref[...] = zcv
            mnn_ref[...] = mn_t
            mxn_ref[...] = mx_t

        @pl.when(i > 0)
        def _():
            numpv_ref[...] += numpv
            denpv_ref[...] += denpv
            zpv_ref[...] += zpv
            numcv_ref[...] += numcv
            dencv_ref[...] += dencv
            zcv_ref[...] += zcv
            mnn_ref[...] = jnp.minimum(mnn_ref[...], mn_t)
            mxn_ref[...] = jnp.maximum(mxn_ref[...], mx_t)

    @pl.when(i < GRID - 1)
    def _():
        body(masked=False)

    @pl.when(i == GRID - 1)
    def _():
        body(masked=True)


def _layer(e, advp, advc, adpv, adcv, pe, ce, mpv, mcv, mn, mx,
           ga1, ga2, ga3, g1, g2, gb):
    c = lambda *shape: pl.BlockSpec(shape, lambda i: (0,) * len(shape))
    return pl.pallas_call(
        _layer_body,
        grid=(GRID,),
        in_specs=[
            pl.BlockSpec((TILE, EMB), lambda i: (i, 0)),
            pl.BlockSpec((TILE, NP), lambda i: (i, 0)),
            pl.BlockSpec((TILE, NC), lambda i: (i, 0)),
            pl.BlockSpec((NP, TILE), lambda i: (0, i)),
            pl.BlockSpec((NC, TILE), lambda i: (0, i)),
            c(NP, EMB), c(NC, EMB), c(NP, 1), c(NC, 1), c(1, 1), c(1, 1),
            c(EMB, EMB), c(EMB, EMB), c(EMB, EMB), c(EMB, EMB), c(EMB, EMB),
            c(1, EMB),
        ],
        out_specs=[
            pl.BlockSpec((TILE, EMB), lambda i: (i, 0)),
            c(NP, EMB), c(NP, 1), c(NP, 1),
            c(NC, EMB), c(NC, 1), c(NC, 1),
            c(1, 1), c(1, 1),
        ],
        out_shape=[
            jax.ShapeDtypeStruct((N, EMB), F32),
            jax.ShapeDtypeStruct((NP, EMB), F32),
            jax.ShapeDtypeStruct((NP, 1), F32),
            jax.ShapeDtypeStruct((NP, 1), F32),
            jax.ShapeDtypeStruct((NC, EMB), F32),
            jax.ShapeDtypeStruct((NC, 1), F32),
            jax.ShapeDtypeStruct((NC, 1), F32),
            jax.ShapeDtypeStruct((1, 1), F32),
            jax.ShapeDtypeStruct((1, 1), F32),
        ],
    )(e, advp, advc, adpv, adcv, pe, ce, mpv, mcv, mn, mx,
      ga1, ga2, ga3, g1, g2, gb)


def _finalize_body(pe_ref, ce_ref, numpv_ref, denpv_ref, zpv_ref,
                   numcv_ref, dencv_ref, zcv_ref, adpc_ref, adcp_ref,
                   mpc_ref, mcp_ref,
                   pa1_ref, pa2_ref, pa3_ref, p1_ref, p2_ref, pb_ref,
                   ca1_ref, ca2_ref, ca3_ref, c1_ref, c2_ref, cb_ref,
                   price_ref, cate_ref):
    pe = pe_ref[...]
    ce = ce_ref[...]
    e_pv = numpv_ref[...] / (denpv_ref[...] + 1e-8 * zpv_ref[...])
    e_cv = numcv_ref[...] / (dencv_ref[...] + 1e-8 * zcv_ref[...])

    def intra_small(adj, m, emb2):
        s = _rowsum_lanes(emb2)
        mn = jnp.min(s, axis=(0, 1), keepdims=True)
        mx = jnp.max(s, axis=(0, 1), keepdims=True)
        shift = jnp.where(m >= 0.0, m * mx, m * mn)
        p = jnp.exp(m * s - shift)
        w = adj * p
        num = _bdot(w, emb2)
        den = jnp.sum(w, axis=1, keepdims=True)
        z = jnp.sum(p, axis=1, keepdims=True)
        return num / (den + 1e-8 * z)

    e_pc = intra_small(adpc_ref[...], mpc_ref[...], ce)
    e_cp = intra_small(adcp_ref[...], mcp_ref[...], pe)

    gp = jax.nn.sigmoid(_bdot(pe, pa1_ref[...]) + _bdot(e_pv, pa2_ref[...])
                        + _bdot(e_pc, pa3_ref[...]) + _bdot(e_pv, p1_ref[...])
                        + _bdot(e_pc, p2_ref[...]) + pb_ref[...])
    price_ref[...] = pe + gp * e_pv + (1.0 - gp) * e_pc
    gc = jax.nn.sigmoid(_bdot(ce, ca1_ref[...]) + _bdot(e_cp, ca2_ref[...])
                        + _bdot(e_cv, ca3_ref[...]) + _bdot(e_cp, c1_ref[...])
                        + _bdot(e_cv, c2_ref[...]) + cb_ref[...])
    cate_ref[...] = ce + gc * e_cp + (1.0 - gc) * e_cv


def _finalize(pe, ce, numpv, denpv, zpv, numcv, dencv, zcv, adpc, adcp,
              mpc, mcp, pw, cw):
    return pl.pallas_call(
        _finalize_body,
        out_shape=[jax.ShapeDtypeStruct((NP, EMB), F32),
                   jax.ShapeDtypeStruct((NC, EMB), F32)],
    )(pe, ce, numpv, denpv, zpv, numcv, dencv, zcv, adpc, adcp,
      mpc, mcp, *pw, *cw)


def kernel(adjacency, adjacency_pv, adjacency_vp, adjacency_pc, adjacency_cp,
           adjacency_cv, adjacency_vc, embedding, pri_emb, cate_emb,
           single_basket, session_basket, mat_pv, mat_pc, mat_cp, mat_cv,
           W_aogi, b_aogi, W_bgi1, b_bgi1, W_bgi2, b_bgi2,
           W_aogp, b_aogp, W_bgp1, b_bgp1, W_bgp2, b_bgp2,
           W_aogc, b_aogc, W_bgc1, b_bgc1, W_bgc2, b_bgc2):
    def split(Wa, ba, W1, b1, W2, b2):
        return (Wa[:EMB], Wa[EMB:2 * EMB], Wa[2 * EMB:], W1, W2,
                (ba + b1 + b2)[None, :])

    gi = split(W_aogi, b_aogi, W_bgi1, b_bgi1, W_bgi2, b_bgi2)
    gp = split(W_aogp, b_aogp, W_bgp1, b_bgp1, W_bgp2, b_bgp2)
    gc = split(W_aogc, b_aogc, W_bgc1, b_bgc1, W_bgc2, b_bgc2)

    rd = lambda m: m.astype(BF16).astype(F32)
    mpv, mpc, mcp, mcv = rd(mat_pv), rd(mat_pc), rd(mat_cp), rd(mat_cv)

    e, pe, ce = embedding, pri_emb, cate_emb
    mn, mx = _prepass(e)
    for _ in range(2):
        (item, numpv, denpv, zpv, numcv, dencv, zcv, mn, mx) = _layer(
            e, adjacency_vp, adjacency_vc, adjacency_pv, adjacency_cv,
            pe, ce, mpv, mcv, mn, mx, *gi)
        price, cate = _finalize(pe, ce, numpv, denpv, zpv, numcv, dencv, zcv,
                                adjacency_pc, adjacency_cp, mpc, mcp, gp, gc)
        e, pe, ce = item, price, cate
    return (e, pe, ce)


# speedup vs baseline: 1.1289x; 1.0015x over previous
import functools

import jax
import jax.numpy as jnp
from jax.experimental import pallas as pl

EMB = 128
NP = 100
NC = 500
N = 50000
TILE = 2048
GRID = -(-N // TILE)
F32 = jnp.float32
BF16 = jnp.bfloat16


def _bdot(a, b):
    return jax.lax.dot_general(a.astype(BF16), b.astype(BF16),
                               (((1,), (0,)), ((), ())),
                               preferred_element_type=F32)


def _rowsum_lanes(x):
    ones = jnp.ones((1, EMB), BF16)
    return jax.lax.dot_general(ones, x.astype(BF16), (((1,), (1,)), ((), ())),
                               preferred_element_type=F32)


def _masks(i):
    rows = jax.lax.broadcasted_iota(jnp.int32, (TILE, 1), 0) + i * TILE
    lanes = jax.lax.broadcasted_iota(jnp.int32, (1, TILE), 1) + i * TILE
    return rows < N, lanes < N


def _prepass_body(e_ref, mn_ref, mx_ref):
    i = pl.program_id(0)
    rmask, lmask = _masks(i)
    s = _rowsum_lanes(jnp.where(rmask, e_ref[...], 0.0))
    mn_t = jnp.min(jnp.where(lmask, s, jnp.inf), axis=(0, 1), keepdims=True)
    mx_t = jnp.max(jnp.where(lmask, s, -jnp.inf), axis=(0, 1), keepdims=True)

    @pl.when(i == 0)
    def _():
        mn_ref[...] = mn_t
        mx_ref[...] = mx_t

    @pl.when(i > 0)
    def _():
        mn_ref[...] = jnp.minimum(mn_ref[...], mn_t)
        mx_ref[...] = jnp.maximum(mx_ref[...], mx_t)


def _prepass(e):
    return pl.pallas_call(
        _prepass_body,
        grid=(GRID,),
        in_specs=[pl.BlockSpec((TILE, EMB), lambda i: (i, 0))],
        out_specs=[pl.BlockSpec((1, 1), lambda i: (0, 0)),
                   pl.BlockSpec((1, 1), lambda i: (0, 0))],
        out_shape=[jax.ShapeDtypeStruct((1, 1), F32),
                   jax.ShapeDtypeStruct((1, 1), F32)],
    )(e)


def _layer_body(compute_next, e_ref, advp_ref, advc_ref, adpv_ref, adcv_ref,
                pe_ref, ce_ref, mpv_ref, mcv_ref, mn_ref, mx_ref,
                ga1_ref, ga2_ref, ga3_ref, g1_ref, g2_ref, gb_ref,
                item_ref, numpv_ref, denpv_ref, zpv_ref,
                numcv_ref, dencv_ref, zcv_ref, mnn_ref, mxn_ref):
    i = pl.program_id(0)

    def body(masked):
        if masked:
            rmask, lmask = _masks(i)
            e = jnp.where(rmask, e_ref[...], 0.0)
        else:
            e = e_ref[...]
        evp = _bdot(advp_ref[...], pe_ref[...])
        evc = _bdot(advc_ref[...], ce_ref[...])
        garg = (_bdot(e, ga1_ref[...]) + _bdot(evp, ga2_ref[...])
                + _bdot(evc, ga3_ref[...]) + _bdot(evp, g1_ref[...])
                + _bdot(evc, g2_ref[...]) + gb_ref[...])
        g = jax.nn.sigmoid(garg)
        item = e + g * evp + (1.0 - g) * evc
        item_ref[...] = item

        s = _rowsum_lanes(e)
        mn_s = mn_ref[...]
        mx_s = mx_ref[...]

        def intra_part(m, adj):
            shift = jnp.where(m >= 0.0, m * mx_s, m * mn_s)
            p = jnp.exp(m * s - shift)
            if masked:
                p = jnp.where(lmask, p, 0.0)
                w = jnp.where(lmask, adj * p, 0.0)
            else:
                w = adj * p
            num = _bdot(w, e)
            den = jnp.sum(w, axis=1, keepdims=True)
            z = jnp.sum(p, axis=1, keepdims=True)
            return num, den, z

        numpv, denpv, zpv = intra_part(mpv_ref[...], adpv_ref[...])
        numcv, dencv, zcv = intra_part(mcv_ref[...], adcv_ref[...])

        if compute_next:
            s_next = _rowsum_lanes(item)
            if masked:
                mn_t = jnp.min(jnp.where(lmask, s_next, jnp.inf),
                               axis=(0, 1), keepdims=True)
                mx_t = jnp.max(jnp.where(lmask, s_next, -jnp.inf),
                               axis=(0, 1), keepdims=True)
            else:
                mn_t = jnp.min(s_next, axis=(0, 1), keepdims=True)
                mx_t = jnp.max(s_next, axis=(0, 1), keepdims=True)
        else:
            mn_t = jnp.zeros((1, 1), F32)
            mx_t = jnp.zeros((1, 1), F32)

        @pl.when(i == 0)
        def _():
            numpv_ref[...] = numpv
            denpv_ref[...] = denpv
            zpv_ref[...] = zpv
            numcv_ref[...] = numcv
            dencv_ref[...] = dencv
            zcv_ref[...] = zcv
            mnn_ref[...] = mn_t
            mxn_ref[...] = mx_t

        @pl.when(i > 0)
        def _():
            numpv_ref[...] += numpv
            denpv_ref[...] += denpv
            zpv_ref[...] += zpv
            numcv_ref[...] += numcv
            dencv_ref[...] += dencv
            zcv_ref[...] += zcv
            mnn_ref[...] = jnp.minimum(mnn_ref[...], mn_t)
            mxn_ref[...] = jnp.maximum(mxn_ref[...], mx_t)

    @pl.when(i < GRID - 1)
    def _():
        body(masked=False)

    @pl.when(i == GRID - 1)
    def _():
        body(masked=True)


def _layer(e, advp, advc, adpv, adcv, pe, ce, mpv, mcv, mn, mx,
           ga1, ga2, ga3, g1, g2, gb, compute_next):
    c = lambda *shape: pl.BlockSpec(shape, lambda i: (0,) * len(shape))
    return pl.pallas_call(
        functools.partial(_layer_body, compute_next),
        grid=(GRID,),
        in_specs=[
            pl.BlockSpec((TILE, EMB), lambda i: (i, 0)),
            pl.BlockSpec((TILE, NP), lambda i: (i, 0)),
            pl.BlockSpec((TILE, NC), lambda i: (i, 0)),
            pl.BlockSpec((NP, TILE), lambda i: (0, i)),
            pl.BlockSpec((NC, TILE), lambda i: (0, i)),
            c(NP, EMB), c(NC, EMB), c(NP, 1), c(NC, 1), c(1, 1), c(1, 1),
            c(EMB, EMB), c(EMB, EMB), c(EMB, EMB), c(EMB, EMB), c(EMB, EMB),
            c(1, EMB),
        ],
        out_specs=[
            pl.BlockSpec((TILE, EMB), lambda i: (i, 0)),
            c(NP, EMB), c(NP, 1), c(NP, 1),
            c(NC, EMB), c(NC, 1), c(NC, 1),
            c(1, 1), c(1, 1),
        ],
        out_shape=[
            jax.ShapeDtypeStruct((N, EMB), F32),
            jax.ShapeDtypeStruct((NP, EMB), F32),
            jax.ShapeDtypeStruct((NP, 1), F32),
            jax.ShapeDtypeStruct((NP, 1), F32),
            jax.ShapeDtypeStruct((NC, EMB), F32),
            jax.ShapeDtypeStruct((NC, 1), F32),
            jax.ShapeDtypeStruct((NC, 1), F32),
            jax.ShapeDtypeStruct((1, 1), F32),
            jax.ShapeDtypeStruct((1, 1), F32),
        ],
    )(e, advp, advc, adpv, adcv, pe, ce, mpv, mcv, mn, mx,
      ga1, ga2, ga3, g1, g2, gb)


def _finalize_body(pe_ref, ce_ref, numpv_ref, denpv_ref, zpv_ref,
                   numcv_ref, dencv_ref, zcv_ref, adpc_ref, adcp_ref,
                   mpc_ref, mcp_ref,
                   pa1_ref, pa2_ref, pa3_ref, p1_ref, p2_ref, pb_ref,
                   ca1_ref, ca2_ref, ca3_ref, c1_ref, c2_ref, cb_ref,
                   price_ref, cate_ref):
    pe = pe_ref[...]
    ce = ce_ref[...]
    e_pv = numpv_ref[...] / (denpv_ref[...] + 1e-8 * zpv_ref[...])
    e_cv = numcv_ref[...] / (dencv_ref[...] + 1e-8 * zcv_ref[...])

    def intra_small(adj, m, emb2):
        s = _rowsum_lanes(emb2)
        mn = jnp.min(s, axis=(0, 1), keepdims=True)
        mx = jnp.max(s, axis=(0, 1), keepdims=True)
        shift = jnp.where(m >= 0.0, m * mx, m * mn)
        p = jnp.exp(m * s - shift)
        w = adj * p
        num = _bdot(w, emb2)
        den = jnp.sum(w, axis=1, keepdims=True)
        z = jnp.sum(p, axis=1, keepdims=True)
        return num / (den + 1e-8 * z)

    e_pc = intra_small(adpc_ref[...], mpc_ref[...], ce)
    e_cp = intra_small(adcp_ref[...], mcp_ref[...], pe)

    gp = jax.nn.sigmoid(_bdot(pe, pa1_ref[...]) + _bdot(e_pv, pa2_ref[...])
                        + _bdot(e_pc, pa3_ref[...]) + _bdot(e_pv, p1_ref[...])
                        + _bdot(e_pc, p2_ref[...]) + pb_ref[...])
    price_ref[...] = pe + gp * e_pv + (1.0 - gp) * e_pc
    gc = jax.nn.sigmoid(_bdot(ce, ca1_ref[...]) + _bdot(e_cp, ca2_ref[...])
                        + _bdot(e_cv, ca3_ref[...]) + _bdot(e_cp, c1_ref[...])
                        + _bdot(e_cv, c2_ref[...]) + cb_ref[...])
    cate_ref[...] = ce + gc * e_cp + (1.0 - gc) * e_cv


def _finalize(pe, ce, numpv, denpv, zpv, numcv, dencv, zcv, adpc, adcp,
              mpc, mcp, pw, cw):
    return pl.pallas_call(
        _finalize_body,
        out_shape=[jax.ShapeDtypeStruct((NP, EMB), F32),
                   jax.ShapeDtypeStruct((NC, EMB), F32)],
    )(pe, ce, numpv, denpv, zpv, numcv, dencv, zcv, adpc, adcp,
      mpc, mcp, *pw, *cw)


def kernel(adjacency, adjacency_pv, adjacency_vp, adjacency_pc, adjacency_cp,
           adjacency_cv, adjacency_vc, embedding, pri_emb, cate_emb,
           single_basket, session_basket, mat_pv, mat_pc, mat_cp, mat_cv,
           W_aogi, b_aogi, W_bgi1, b_bgi1, W_bgi2, b_bgi2,
           W_aogp, b_aogp, W_bgp1, b_bgp1, W_bgp2, b_bgp2,
           W_aogc, b_aogc, W_bgc1, b_bgc1, W_bgc2, b_bgc2):
    def split(Wa, ba, W1, b1, W2, b2):
        return (Wa[:EMB], Wa[EMB:2 * EMB], Wa[2 * EMB:], W1, W2,
                (ba + b1 + b2)[None, :])

    gi = split(W_aogi, b_aogi, W_bgi1, b_bgi1, W_bgi2, b_bgi2)
    gp = split(W_aogp, b_aogp, W_bgp1, b_bgp1, W_bgp2, b_bgp2)
    gc = split(W_aogc, b_aogc, W_bgc1, b_bgc1, W_bgc2, b_bgc2)

    rd = lambda m: m.astype(BF16).astype(F32)
    mpv, mpc, mcp, mcv = rd(mat_pv), rd(mat_pc), rd(mat_cp), rd(mat_cv)

    e, pe, ce = embedding, pri_emb, cate_emb
    mn, mx = _prepass(e)
    for layer in range(2):
        (item, numpv, denpv, zpv, numcv, dencv, zcv, mn, mx) = _layer(
            e, adjacency_vp, adjacency_vc, adjacency_pv, adjacency_cv,
            pe, ce, mpv, mcv, mn, mx, *gi, compute_next=(layer == 0))
        price, cate = _finalize(pe, ce, numpv, denpv, zpv, numcv, dencv, zcv,
                                adjacency_pc, adjacency_cp, mpc, mcp, gp, gc)
        e, pe, ce = item, price, cate
    return (e, pe, ce)


# speedup vs baseline: 1.1862x; 1.0507x over previous
import functools

import jax
import jax.numpy as jnp
from jax.experimental import pallas as pl
from jax.experimental.pallas import tpu as pltpu

EMB = 128
NP = 100
NC = 500
N = 50000
TILE = 2048
GRID = -(-N // TILE)
F32 = jnp.float32
BF16 = jnp.bfloat16


def _bdot(a, b):
    return jax.lax.dot_general(a.astype(BF16), b.astype(BF16),
                               (((1,), (0,)), ((), ())),
                               preferred_element_type=F32)


def _rowsum_lanes(x):
    ones = jnp.ones((1, EMB), BF16)
    return jax.lax.dot_general(ones, x.astype(BF16), (((1,), (1,)), ((), ())),
                               preferred_element_type=F32)


def _masks(i):
    rows = jax.lax.broadcasted_iota(jnp.int32, (TILE, 1), 0) + i * TILE
    lanes = jax.lax.broadcasted_iota(jnp.int32, (1, TILE), 1) + i * TILE
    return rows < N, lanes < N


def _layer_body(e_ref, advp_ref, advc_ref, adpv_ref, adcv_ref,
                pe_ref, ce_ref, mpv_ref, mcv_ref,
                ga1_ref, ga2_ref, ga3_ref, g1_ref, g2_ref, gb_ref,
                item_ref, numpv_ref, denpv_ref, zpv_ref,
                numcv_ref, dencv_ref, zcv_ref,
                shpv_sc, shcv_sc):
    i = pl.program_id(0)

    @pl.when(i == 0)
    def _():
        for ref in (numpv_ref, denpv_ref, zpv_ref, numcv_ref, dencv_ref,
                    zcv_ref):
            ref[...] = jnp.zeros(ref.shape, ref.dtype)
        shpv_sc[...] = jnp.full(shpv_sc.shape, -jnp.inf, F32)
        shcv_sc[...] = jnp.full(shcv_sc.shape, -jnp.inf, F32)

    def body(masked):
        if masked:
            rmask, lmask = _masks(i)
            e = jnp.where(rmask, e_ref[...], 0.0)
        else:
            e = e_ref[...]
        evp = _bdot(advp_ref[...], pe_ref[...])
        evc = _bdot(advc_ref[...], ce_ref[...])
        garg = (_bdot(e, ga1_ref[...]) + _bdot(evp, ga2_ref[...])
                + _bdot(evc, ga3_ref[...]) + _bdot(evp, g1_ref[...])
                + _bdot(evc, g2_ref[...]) + gb_ref[...])
        g = jax.nn.sigmoid(garg)
        item = e + g * evp + (1.0 - g) * evc
        item_ref[...] = item

        s = _rowsum_lanes(e)
        if masked:
            smx = jnp.max(jnp.where(lmask, s, -jnp.inf), axis=(0, 1),
                          keepdims=True)
            smn = jnp.min(jnp.where(lmask, s, jnp.inf), axis=(0, 1),
                          keepdims=True)
        else:
            smx = jnp.max(s, axis=(0, 1), keepdims=True)
            smn = jnp.min(s, axis=(0, 1), keepdims=True)

        def intra_part(m, adj, num_ref, den_ref, z_ref, sh_ref):
            sh_t = jnp.where(m >= 0.0, m * smx, m * smn)
            p = jnp.exp(m * s - sh_t)
            if masked:
                p = jnp.where(lmask, p, 0.0)
                w = jnp.where(lmask, adj * p, 0.0)
            else:
                w = adj * p
            sh_old = sh_ref[...]
            sh_new = jnp.maximum(sh_old, sh_t)
            alpha = jnp.exp(sh_old - sh_new)
            beta = jnp.exp(sh_t - sh_new)
            num_ref[...] = num_ref[...] * alpha + _bdot(w, e) * beta
            den_ref[...] = (den_ref[...] * alpha
                            + jnp.sum(w, axis=1, keepdims=True) * beta)
            z_ref[...] = (z_ref[...] * alpha
                          + jnp.sum(p, axis=1, keepdims=True) * beta)
            sh_ref[...] = sh_new

        intra_part(mpv_ref[...], adpv_ref[...],
                   numpv_ref, denpv_ref, zpv_ref, shpv_sc)
        intra_part(mcv_ref[...], adcv_ref[...],
                   numcv_ref, dencv_ref, zcv_ref, shcv_sc)

    @pl.when(i < GRID - 1)
    def _():
        body(masked=False)

    @pl.when(i == GRID - 1)
    def _():
        body(masked=True)


def _layer(e, advp, advc, adpv, adcv, pe, ce, mpv, mcv,
           ga1, ga2, ga3, g1, g2, gb):
    c = lambda *shape: pl.BlockSpec(shape, lambda i: (0,) * len(shape))
    return pl.pallas_call(
        _layer_body,
        grid=(GRID,),
        in_specs=[
            pl.BlockSpec((TILE, EMB), lambda i: (i, 0)),
            pl.BlockSpec((TILE, NP), lambda i: (i, 0)),
            pl.BlockSpec((TILE, NC), lambda i: (i, 0)),
            pl.BlockSpec((NP, TILE), lambda i: (0, i)),
            pl.BlockSpec((NC, TILE), lambda i: (0, i)),
            c(NP, EMB), c(NC, EMB), c(NP, 1), c(NC, 1),
            c(EMB, EMB), c(EMB, EMB), c(EMB, EMB), c(EMB, EMB), c(EMB, EMB),
            c(1, EMB),
        ],
        out_specs=[
            pl.BlockSpec((TILE, EMB), lambda i: (i, 0)),
            c(NP, EMB), c(NP, 1), c(NP, 1),
            c(NC, EMB), c(NC, 1), c(NC, 1),
        ],
        out_shape=[
            jax.ShapeDtypeStruct((N, EMB), F32),
            jax.ShapeDtypeStruct((NP, EMB), F32),
            jax.ShapeDtypeStruct((NP, 1), F32),
            jax.ShapeDtypeStruct((NP, 1), F32),
            jax.ShapeDtypeStruct((NC, EMB), F32),
            jax.ShapeDtypeStruct((NC, 1), F32),
            jax.ShapeDtypeStruct((NC, 1), F32),
        ],
        scratch_shapes=[pltpu.VMEM((NP, 1), F32), pltpu.VMEM((NC, 1), F32)],
    )(e, advp, advc, adpv, adcv, pe, ce, mpv, mcv,
      ga1, ga2, ga3, g1, g2, gb)


def _finalize_body(pe_ref, ce_ref, numpv_ref, denpv_ref, zpv_ref,
                   numcv_ref, dencv_ref, zcv_ref, adpc_ref, adcp_ref,
                   mpc_ref, mcp_ref,
                   pa1_ref, pa2_ref, pa3_ref, p1_ref, p2_ref, pb_ref,
                   ca1_ref, ca2_ref, ca3_ref, c1_ref, c2_ref, cb_ref,
                   price_ref, cate_ref):
    pe = pe_ref[...]
    ce = ce_ref[...]
    e_pv = numpv_ref[...] / (denpv_ref[...] + 1e-8 * zpv_ref[...])
    e_cv = numcv_ref[...] / (dencv_ref[...] + 1e-8 * zcv_ref[...])

    def intra_small(adj, m, emb2):
        s = _rowsum_lanes(emb2)
        mn = jnp.min(s, axis=(0, 1), keepdims=True)
        mx = jnp.max(s, axis=(0, 1), keepdims=True)
        shift = jnp.where(m >= 0.0, m * mx, m * mn)
        p = jnp.exp(m * s - shift)
        w = adj * p
        num = _bdot(w, emb2)
        den = jnp.sum(w, axis=1, keepdims=True)
        z = jnp.sum(p, axis=1, keepdims=True)
        return num / (den + 1e-8 * z)

    e_pc = intra_small(adpc_ref[...], mpc_ref[...], ce)
    e_cp = intra_small(adcp_ref[...], mcp_ref[...], pe)

    gp = jax.nn.sigmoid(_bdot(pe, pa1_ref[...]) + _bdot(e_pv, pa2_ref[...])
                        + _bdot(e_pc, pa3_ref[...]) + _bdot(e_pv, p1_ref[...])
                        + _bdot(e_pc, p2_ref[...]) + pb_ref[...])
    price_ref[...] = pe + gp * e_pv + (1.0 - gp) * e_pc
    gc = jax.nn.sigmoid(_bdot(ce, ca1_ref[...]) + _bdot(e_cp, ca2_ref[...])
                        + _bdot(e_cv, ca3_ref[...]) + _bdot(e_cp, c1_ref[...])
                        + _bdot(e_cv, c2_ref[...]) + cb_ref[...])
    cate_ref[...] = ce + gc * e_cp + (1.0 - gc) * e_cv


def _finalize(pe, ce, numpv, denpv, zpv, numcv, dencv, zcv, adpc, adcp,
              mpc, mcp, pw, cw):
    return pl.pallas_call(
        _finalize_body,
        out_shape=[jax.ShapeDtypeStruct((NP, EMB), F32),
                   jax.ShapeDtypeStruct((NC, EMB), F32)],
    )(pe, ce, numpv, denpv, zpv, numcv, dencv, zcv, adpc, adcp,
      mpc, mcp, *pw, *cw)


def kernel(adjacency, adjacency_pv, adjacency_vp, adjacency_pc, adjacency_cp,
           adjacency_cv, adjacency_vc, embedding, pri_emb, cate_emb,
           single_basket, session_basket, mat_pv, mat_pc, mat_cp, mat_cv,
           W_aogi, b_aogi, W_bgi1, b_bgi1, W_bgi2, b_bgi2,
           W_aogp, b_aogp, W_bgp1, b_bgp1, W_bgp2, b_bgp2,
           W_aogc, b_aogc, W_bgc1, b_bgc1, W_bgc2, b_bgc2):
    def split(Wa, ba, W1, b1, W2, b2):
        return (Wa[:EMB], Wa[EMB:2 * EMB], Wa[2 * EMB:], W1, W2,
                (ba + b1 + b2)[None, :])

    gi = split(W_aogi, b_aogi, W_bgi1, b_bgi1, W_bgi2, b_bgi2)
    gp = split(W_aogp, b_aogp, W_bgp1, b_bgp1, W_bgp2, b_bgp2)
    gc = split(W_aogc, b_aogc, W_bgc1, b_bgc1, W_bgc2, b_bgc2)

    rd = lambda m: m.astype(BF16).astype(F32)
    mpv, mpc, mcp, mcv = rd(mat_pv), rd(mat_pc), rd(mat_cp), rd(mat_cv)

    e, pe, ce = embedding, pri_emb, cate_emb
    for layer in range(2):
        (item, numpv, denpv, zpv, numcv, dencv, zcv) = _layer(
            e, adjacency_vp, adjacency_vc, adjacency_pv, adjacency_cv,
            pe, ce, mpv, mcv, *gi)
        price, cate = _finalize(pe, ce, numpv, denpv, zpv, numcv, dencv, zcv,
                                adjacency_pc, adjacency_cp, mpc, mcp, gp, gc)
        e, pe, ce = item, price, cate
    return (e, pe, ce)
